# Initial kernel scaffold; baseline (speedup 1.0000x reference)
import functools

import jax
import jax.numpy as jnp
from jax import lax
from jax.experimental import pallas as pl
from jax.experimental.pallas import tpu as pltpu
from jax.experimental.pallas import tpu_sc as plsc

NUM_USERS = 5000
NUM_ITEMS = 5000
N = NUM_USERS + NUM_ITEMS
E = 320000
D = 128
NLAYERS = 3
B = 4096

NC = 2
NS = 16
NW = NC * NS
L = 16

K = 128
E_PAD = ((E + NW * K - 1) // (NW * K)) * (NW * K)
EPW = E_PAD // NW
NB = EPW // K
RPT = N // NS

_SC_MESH = plsc.VectorSubcoreMesh(core_axis_name="c", subcore_axis_name="s")


@functools.partial(
    pl.kernel,
    out_type=jax.ShapeDtypeStruct((NC, N, D), jnp.float32),
    mesh=_SC_MESH,
    scratch_types=[
        pltpu.VMEM((K,), jnp.int32),
        pltpu.VMEM((K,), jnp.int32),
        pltpu.VMEM((K,), jnp.float32),
        pltpu.VMEM((K, D), jnp.float32),
        pltpu.VMEM_SHARED((N, D), jnp.float32),
        pltpu.SemaphoreType.DMA,
    ],
)
def _spmm_sc(f_hbm, src_hbm, dst_hbm, w_hbm, out_hbm,
             src_v, dst_v, w_v, rows_v, accum, sem):
    cid = lax.axis_index("c")
    sid = lax.axis_index("s")

    def zero_body(k, _):
        for j in range(D // L):
            rows_v[k, pl.ds(j * L, L)] = jnp.zeros((L,), jnp.float32)
        return 0
    lax.fori_loop(0, K, zero_body, 0)
    row0 = sid * RPT
    nfull = RPT // K
    for i in range(nfull):
        pltpu.sync_copy(rows_v, accum.at[pl.ds(row0 + i * K, K)])
    rem = RPT - nfull * K
    pltpu.sync_copy(rows_v.at[pl.ds(0, rem)],
                    accum.at[pl.ds(row0 + nfull * K, rem)])
    plsc.subcore_barrier()

    estart = (cid * NS + sid) * EPW

    def batch_body(b, _):
        base = estart + b * K
        pltpu.sync_copy(src_hbm.at[pl.ds(base, K)], src_v)
        pltpu.sync_copy(dst_hbm.at[pl.ds(base, K)], dst_v)
        pltpu.sync_copy(w_hbm.at[pl.ds(base, K)], w_v)
        pltpu.async_copy(f_hbm.at[src_v], rows_v, sem).wait()

        def edge_body(k, _):
            wk = w_v[k]
            for j in range(D // L):
                sl = pl.ds(j * L, L)
                rows_v[k, sl] = rows_v[k, sl] * wk
            return 0
        lax.fori_loop(0, K, edge_body, 0)
        pltpu.sync_copy(rows_v, accum.at[dst_v], add=True)
        return 0

    lax.fori_loop(0, NB, batch_body, 0)
    plsc.subcore_barrier()

    pltpu.sync_copy(accum.at[pl.ds(row0, RPT)],
                    out_hbm.at[cid, pl.ds(row0, RPT)])


RB = 1000


def _dense_body(lf_ref, f_ref, wl_ref, wi_ref, b_ref, o_ref):
    lf = lf_ref[0] + lf_ref[1]
    f = f_ref[...]
    a = lf + f
    m = lf * f
    h = lax.dot_general(a, wl_ref[...], (((1,), (1,)), ((), ())),
                        precision=lax.Precision.HIGHEST,
                        preferred_element_type=jnp.float32)
    h = h + lax.dot_general(m, wi_ref[...], (((1,), (1,)), ((), ())),
                            precision=lax.Precision.HIGHEST,
                            preferred_element_type=jnp.float32)
    h = h + b_ref[...]
    h = jnp.where(h >= 0, h, 0.01 * h)
    nrm = jnp.sqrt(jnp.sum(h * h, axis=1, keepdims=True))
    o_ref[...] = h / jnp.maximum(nrm, 1e-12)


def _dense_tc(parts, f, wl, wi, b):
    return pl.pallas_call(
        _dense_body,
        grid=(N // RB,),
        in_specs=[
            pl.BlockSpec((NC, RB, D), lambda i: (0, i, 0)),
            pl.BlockSpec((RB, D), lambda i: (i, 0)),
            pl.BlockSpec((D, D), lambda i: (0, 0)),
            pl.BlockSpec((D, D), lambda i: (0, 0)),
            pl.BlockSpec((1, D), lambda i: (0, 0)),
        ],
        out_specs=pl.BlockSpec((RB, D), lambda i: (i, 0)),
        out_shape=jax.ShapeDtypeStruct((N, D), jnp.float32),
    )(parts, f, wl, wi, b)


PPW = B // NW


@functools.partial(
    pl.kernel,
    out_type=jax.ShapeDtypeStruct((B,), jnp.float32),
    mesh=_SC_MESH,
    scratch_types=[
        pltpu.VMEM((PPW,), jnp.int32),
        pltpu.VMEM((PPW,), jnp.int32),
        pltpu.VMEM((PPW, D), jnp.float32),
        pltpu.VMEM((PPW, D), jnp.float32),
        pltpu.VMEM((PPW, L), jnp.float32),
        pltpu.VMEM((PPW,), jnp.float32),
        pltpu.SemaphoreType.DMA,
        pltpu.SemaphoreType.DMA,
    ],
)
def _logits_sc(f0, f1, f2, f3, uidx_hbm, iidx_hbm, out_hbm,
               uidx_v, iidx_v, urows, irows, acc, out_v, sem_u, sem_i):
    cid = lax.axis_index("c")
    sid = lax.axis_index("s")
    base = (cid * NS + sid) * PPW
    pltpu.sync_copy(uidx_hbm.at[pl.ds(base, PPW)], uidx_v)
    pltpu.sync_copy(iidx_hbm.at[pl.ds(base, PPW)], iidx_v)

    def zero_body(p, _):
        acc[p, :] = jnp.zeros((L,), jnp.float32)
        return 0
    lax.fori_loop(0, PPW, zero_body, 0)

    for arr in (f0, f1, f2, f3):
        cp_u = pltpu.async_copy(arr.at[uidx_v], urows, sem_u)
        cp_i = pltpu.async_copy(arr.at[iidx_v], irows, sem_i)
        cp_u.wait()
        cp_i.wait()

        def pair_body(p, _):
            a = acc[p, :]
            for j in range(D // L):
                sl = pl.ds(j * L, L)
                a = a + urows[p, sl] * irows[p, sl]
            acc[p, :] = a
            return 0
        lax.fori_loop(0, PPW, pair_body, 0)

    def reduce_body(p, _):
        out_v[p] = jnp.sum(acc[p, :])
        return 0
    lax.fori_loop(0, PPW, reduce_body, 0)
    pltpu.sync_copy(out_v, out_hbm.at[pl.ds(base, PPW)])


def kernel(userIdx, itemIdx, edge_index, edge_weight, uEmbd, iEmbd,
           Wlin, blin, Wint, bint):
    f0 = jnp.concatenate([uEmbd, iEmbd], axis=0)
    pad = E_PAD - E
    src = jnp.pad(edge_index[0], (0, pad))
    dst = jnp.pad(edge_index[1], (0, pad))
    w = jnp.pad(edge_weight, (0, pad))
    iidx2 = itemIdx + NUM_USERS

    f = f0
    fs = [f0]
    for l in range(NLAYERS):
        parts = _spmm_sc(f, src, dst, w)
        b_l = (blin[l] + bint[l]).reshape(1, D)
        f = _dense_tc(parts, f, Wlin[l], Wint[l], b_l)
        fs.append(f)
    return _logits_sc(fs[0], fs[1], fs[2], fs[3], userIdx, iidx2)


# speedup vs baseline: 3.2578x; 3.2578x over previous
import functools

import jax
import jax.numpy as jnp
from jax import lax
from jax.experimental import pallas as pl
from jax.experimental.pallas import tpu as pltpu
from jax.experimental.pallas import tpu_sc as plsc

NUM_USERS = 5000
NUM_ITEMS = 5000
N = NUM_USERS + NUM_ITEMS
E = 320000
D = 128
NLAYERS = 3
B = 4096

NC = 2
NS = 16
NW = NC * NS
L = 16

K = 128
E_PAD = ((E + NW * K - 1) // (NW * K)) * (NW * K)
EPW = E_PAD // NW
NB = EPW // K
N_PAD = 10240
RPT = N_PAD // NS

_SC_MESH = plsc.VectorSubcoreMesh(core_axis_name="c", subcore_axis_name="s")


@functools.partial(
    pl.kernel,
    out_type=jax.ShapeDtypeStruct((NC, N_PAD, D), jnp.float32),
    mesh=_SC_MESH,
    scratch_types=[
        pltpu.VMEM((K,), jnp.int32),
        pltpu.VMEM((K,), jnp.int32),
        pltpu.VMEM((K,), jnp.float32),
        pltpu.VMEM((K, D), jnp.float32),
        pltpu.VMEM_SHARED((N_PAD, D), jnp.float32),
        pltpu.SemaphoreType.DMA,
    ],
)
def _spmm_sc(f_hbm, src_hbm, dst_hbm, w_hbm, out_hbm,
             src_v, dst_v, w_v, rows_v, accum, sem):
    cid = lax.axis_index("c")
    sid = lax.axis_index("s")

    def zero_body(k, _):
        for j in range(D // L):
            rows_v[k, pl.ds(j * L, L)] = jnp.zeros((L,), jnp.float32)
        return 0
    lax.fori_loop(0, K, zero_body, 0)
    row0 = sid * RPT
    for i in range(RPT // K):
        pltpu.sync_copy(rows_v, accum.at[pl.ds(row0 + i * K, K)])
    plsc.subcore_barrier()

    estart = (cid * NS + sid) * EPW

    def batch_body(b, _):
        base = estart + b * K
        pltpu.sync_copy(src_hbm.at[pl.ds(base, K)], src_v)
        pltpu.sync_copy(dst_hbm.at[pl.ds(base, K)], dst_v)
        pltpu.sync_copy(w_hbm.at[pl.ds(base, K)], w_v)
        pltpu.async_copy(f_hbm.at[src_v], rows_v, sem).wait()

        def group_body(g, _):
            wv = w_v[pl.ds(g * L, L)]
            for lane in range(L):
                wk = wv[lane]
                k = g * L + lane
                for j in range(D // L):
                    sl = pl.ds(j * L, L)
                    rows_v[k, sl] = rows_v[k, sl] * wk
            return 0
        lax.fori_loop(0, K // L, group_body, 0)
        pltpu.sync_copy(rows_v, accum.at[dst_v], add=True)
        return 0

    lax.fori_loop(0, NB, batch_body, 0)
    plsc.subcore_barrier()

    pltpu.sync_copy(accum.at[pl.ds(row0, RPT)],
                    out_hbm.at[cid, pl.ds(row0, RPT)])


RB = 1000


def _dense_body(lf_ref, f_ref, wl_ref, wi_ref, b_ref, o_ref):
    lf = lf_ref[0] + lf_ref[1]
    f = f_ref[...]
    a = lf + f
    m = lf * f
    h = lax.dot_general(a, wl_ref[...], (((1,), (1,)), ((), ())),
                        precision=lax.Precision.HIGHEST,
                        preferred_element_type=jnp.float32)
    h = h + lax.dot_general(m, wi_ref[...], (((1,), (1,)), ((), ())),
                            precision=lax.Precision.HIGHEST,
                            preferred_element_type=jnp.float32)
    h = h + b_ref[...]
    h = jnp.where(h >= 0, h, 0.01 * h)
    nrm = jnp.sqrt(jnp.sum(h * h, axis=1, keepdims=True))
    o_ref[...] = h / jnp.maximum(nrm, 1e-12)


def _dense_tc(parts, f, wl, wi, b):
    return pl.pallas_call(
        _dense_body,
        grid=(N // RB,),
        in_specs=[
            pl.BlockSpec((NC, RB, D), lambda i: (0, i, 0)),
            pl.BlockSpec((RB, D), lambda i: (i, 0)),
            pl.BlockSpec((D, D), lambda i: (0, 0)),
            pl.BlockSpec((D, D), lambda i: (0, 0)),
            pl.BlockSpec((1, D), lambda i: (0, 0)),
        ],
        out_specs=pl.BlockSpec((RB, D), lambda i: (i, 0)),
        out_shape=jax.ShapeDtypeStruct((N, D), jnp.float32),
    )(parts, f, wl, wi, b)


PPW = B // NW


@functools.partial(
    pl.kernel,
    out_type=jax.ShapeDtypeStruct((B, L), jnp.float32),
    mesh=_SC_MESH,
    scratch_types=[
        pltpu.VMEM((PPW,), jnp.int32),
        pltpu.VMEM((PPW,), jnp.int32),
        pltpu.VMEM((PPW, D), jnp.float32),
        pltpu.VMEM((PPW, D), jnp.float32),
        pltpu.VMEM((PPW, L), jnp.float32),
        pltpu.SemaphoreType.DMA,
        pltpu.SemaphoreType.DMA,
    ],
)
def _logits_sc(f0, f1, f2, f3, uidx_hbm, iidx_hbm, out_hbm,
               uidx_v, iidx_v, urows, irows, acc, sem_u, sem_i):
    cid = lax.axis_index("c")
    sid = lax.axis_index("s")
    base = (cid * NS + sid) * PPW
    pltpu.sync_copy(uidx_hbm.at[pl.ds(base, PPW)], uidx_v)
    pltpu.sync_copy(iidx_hbm.at[pl.ds(base, PPW)], iidx_v)

    def zero_body(p, _):
        acc[p, :] = jnp.zeros((L,), jnp.float32)
        return 0
    lax.fori_loop(0, PPW, zero_body, 0)

    for arr in (f0, f1, f2, f3):
        cp_u = pltpu.async_copy(arr.at[uidx_v], urows, sem_u)
        cp_i = pltpu.async_copy(arr.at[iidx_v], irows, sem_i)
        cp_u.wait()
        cp_i.wait()

        def pair_body(p, _):
            a = acc[p, :]
            for j in range(D // L):
                sl = pl.ds(j * L, L)
                a = a + urows[p, sl] * irows[p, sl]
            acc[p, :] = a
            return 0
        lax.fori_loop(0, PPW, pair_body, 0)

    pltpu.sync_copy(acc, out_hbm.at[pl.ds(base, PPW)])


def _finish_body(p_ref, o_ref):
    o_ref[...] = jnp.sum(p_ref[...], axis=1, keepdims=True)


def _finish_tc(partials):
    out = pl.pallas_call(
        _finish_body,
        out_shape=jax.ShapeDtypeStruct((B, 1), jnp.float32),
    )(partials)
    return out.reshape(B)


def kernel(userIdx, itemIdx, edge_index, edge_weight, uEmbd, iEmbd,
           Wlin, blin, Wint, bint):
    f0 = jnp.concatenate([uEmbd, iEmbd], axis=0)
    pad = E_PAD - E
    src = jnp.pad(edge_index[0], (0, pad))
    dst = jnp.pad(edge_index[1], (0, pad))
    w = jnp.pad(edge_weight, (0, pad))
    iidx2 = itemIdx + NUM_USERS

    f = f0
    fs = [f0]
    for l in range(NLAYERS):
        parts = _spmm_sc(f, src, dst, w)
        b_l = (blin[l] + bint[l]).reshape(1, D)
        f = _dense_tc(parts, f, Wlin[l], Wint[l], b_l)
        fs.append(f)
    partials = _logits_sc(fs[0], fs[1], fs[2], fs[3], userIdx, iidx2)
    return _finish_tc(partials)
